# Initial kernel scaffold; baseline (speedup 1.0000x reference)
import jax
import jax.numpy as jnp
from jax.experimental import pallas as pl


def kernel(pos, batch, y, params):
    raise NotImplementedError("write your pallas kernel here")


# speedup vs baseline: 6.8609x; 6.8609x over previous
import functools

import jax
import jax.numpy as jnp
from jax.experimental import pallas as pl

B = 8
P = 1024
N1 = 512
N2 = 128
R1SQ = 0.2 * 0.2
R2SQ = 0.4 * 0.4
MAXN = 64
K_CURV = 10
CURV_SCALAR = 10.0

_F32 = jnp.float32


def _nth_smallest(d2, n, hi0, iters):
    rows = d2.shape[0]
    lo0 = jnp.zeros((rows, 1), _F32)
    hi_init = jnp.full((rows, 1), hi0, _F32)
    nf = jnp.float32(n)

    def body(_, c):
        lo, hi = c
        mid = 0.5 * (lo + hi)
        cnt = jnp.sum(jnp.where(d2 <= mid, 1.0, 0.0), axis=1, keepdims=True)
        ge = cnt >= nf
        return jnp.where(ge, lo, mid), jnp.where(ge, mid, hi)

    _, hi = jax.lax.fori_loop(0, iters, body, (lo0, hi_init))
    return hi


def _cov_kernel(pos_ref, cov_ref):
    p = pos_ref[0]
    g = jnp.dot(p, p.T, preferred_element_type=_F32)
    n = jnp.sum(p * p, axis=1)
    d2 = n[:, None] + n[None, :] - 2.0 * g
    t = _nth_smallest(d2, K_CURV, 3.0, 44)
    m = jnp.where(d2 <= t, 1.0, 0.0)
    s1 = jnp.dot(m, p, preferred_element_type=_F32)
    mean = s1 / K_CURV
    x, y, z = p[:, 0], p[:, 1], p[:, 2]

    def s2(col):
        return jnp.sum(m * col[None, :], axis=1) / K_CURV

    c00 = s2(x * x) - mean[:, 0] * mean[:, 0]
    c11 = s2(y * y) - mean[:, 1] * mean[:, 1]
    c22 = s2(z * z) - mean[:, 2] * mean[:, 2]
    c01 = s2(x * y) - mean[:, 0] * mean[:, 1]
    c02 = s2(x * z) - mean[:, 0] * mean[:, 2]
    c12 = s2(y * z) - mean[:, 1] * mean[:, 2]
    cov_ref[0, 0, :] = c00
    cov_ref[0, 1, :] = c11
    cov_ref[0, 2, :] = c22
    cov_ref[0, 3, :] = c01
    cov_ref[0, 4, :] = c02
    cov_ref[0, 5, :] = c12
    cov_ref[0, 6, :] = jnp.zeros((P,), _F32)
    cov_ref[0, 7, :] = jnp.zeros((P,), _F32)


def _curvature_cov(pos3):
    return pl.pallas_call(
        _cov_kernel,
        grid=(B,),
        in_specs=[pl.BlockSpec((1, P, 3), lambda c: (c, 0, 0))],
        out_specs=pl.BlockSpec((1, 8, P), lambda c: (c, 0, 0)),
        out_shape=jax.ShapeDtypeStruct((B, 8, P), _F32),
    )(pos3)


def _smallest_eig_ratio(cov):
    a, b, c = cov[:, 0], cov[:, 1], cov[:, 2]
    d, e, f = cov[:, 3], cov[:, 4], cov[:, 5]
    tr = a + b + c
    q = tr / 3.0
    p1 = d * d + e * e + f * f
    a1, b1, c1 = a - q, b - q, c - q
    p2 = a1 * a1 + b1 * b1 + c1 * c1 + 2.0 * p1
    degen = p2 < 1e-30
    p = jnp.sqrt(jnp.where(degen, 1.0, p2) / 6.0)
    det = (a1 * (b1 * c1 - f * f)
           - d * (d * c1 - f * e)
           + e * (d * f - b1 * e))
    r = jnp.clip(det / (2.0 * p * p * p), -1.0, 1.0)
    phi = jnp.arccos(r) / 3.0
    eig_min = q + 2.0 * p * jnp.cos(phi + 2.0 * jnp.pi / 3.0)
    eig_min = jnp.where(degen, q, eig_min)
    return eig_min / (tr + 1e-12)


def _fps_kernel(pos_ref, wfac_ref, wfac1_ref, pos1_ref, pos2_ref):
    pos = pos_ref[...]
    wfac = wfac_ref[...]
    io_p = jax.lax.broadcasted_iota(jnp.int32, (B, P), 1)
    big = jnp.int32(1 << 30)

    def gather1(idx):
        oh = jnp.where(io_p == idx[:, None], 1.0, 0.0)
        lastp = jnp.sum(oh[:, :, None] * pos, axis=1)
        lastw = jnp.sum(oh * wfac, axis=1)
        return lastp, lastw

    def body1(i, carry):
        mind, idx = carry
        lastp, lastw = gather1(idx)
        pos1_ref[pl.ds(i - 1, 1)] = jnp.transpose(lastp)[None]
        wfac1_ref[pl.ds(i - 1, 1)] = lastw[None]
        diff = pos - lastp[:, None, :]
        dd = jnp.sum(diff * diff, axis=2)
        mind = jnp.minimum(mind, dd)
        score = mind * wfac
        mx = jnp.max(score, axis=1)
        nidx = jnp.min(jnp.where(score == mx[:, None], io_p, big), axis=1)
        return mind, nidx.astype(jnp.int32)

    mind0 = jnp.full((B, P), jnp.inf, _F32)
    idx0 = jnp.zeros((B,), jnp.int32)
    _, idx_last = jax.lax.fori_loop(1, N1, body1, (mind0, idx0))
    lastp, lastw = gather1(idx_last)
    pos1_ref[pl.ds(N1 - 1, 1)] = jnp.transpose(lastp)[None]
    wfac1_ref[pl.ds(N1 - 1, 1)] = lastw[None]

    p1 = pos1_ref[...]
    w1 = wfac1_ref[...]
    io_n = jax.lax.broadcasted_iota(jnp.int32, (N1, B), 0)

    def gather2(idx):
        oh = jnp.where(io_n == idx[None, :], 1.0, 0.0)
        lastp = jnp.sum(oh[:, None, :] * p1, axis=0)
        return lastp

    def body2(i, carry):
        mind, idx = carry
        lastp = gather2(idx)
        pos2_ref[pl.ds(i - 1, 1)] = lastp[None]
        diff = p1 - lastp[None]
        dd = jnp.sum(diff * diff, axis=1)
        mind = jnp.minimum(mind, dd)
        score = mind * w1
        mx = jnp.max(score, axis=0)
        nidx = jnp.min(jnp.where(score == mx[None, :], io_n, big), axis=0)
        return mind, nidx.astype(jnp.int32)

    mind0b = jnp.full((N1, B), jnp.inf, _F32)
    _, idx_last2 = jax.lax.fori_loop(1, N2, body2, (mind0b, idx0))
    pos2_ref[pl.ds(N2 - 1, 1)] = gather2(idx_last2)[None]


def _fps(pos3, curv):
    wfac = 1.0 + CURV_SCALAR * curv
    _, pos1_t, pos2_t = pl.pallas_call(
        _fps_kernel,
        out_shape=(
            jax.ShapeDtypeStruct((N1, B), _F32),
            jax.ShapeDtypeStruct((N1, 3, B), _F32),
            jax.ShapeDtypeStruct((N2, 3, B), _F32),
        ),
    )(pos3, wfac)
    pos1 = jnp.transpose(pos1_t, (2, 0, 1))
    pos2 = jnp.transpose(pos2_t, (2, 0, 1))
    return pos1, pos2


def _sa1_kernel(pos_ref, ctr_ref, w1_ref, b1_ref, w2_ref, b2_ref,
                w3_ref, b3_ref, out_ref):
    p = pos_ref[0]
    q = ctr_ref[0]
    np_ = jnp.sum(p * p, axis=1)
    nq = jnp.sum(q * q, axis=1)
    d2 = nq[:, None] + np_[None, :] - 2.0 * jnp.dot(
        q, p.T, preferred_element_type=_F32)
    t = _nth_smallest(d2, MAXN, R1SQ, 40)
    mask = d2 <= t
    a = jnp.dot(p, w1_ref[...], preferred_element_type=_F32) + b1_ref[...]
    bq = jnp.dot(q, w1_ref[...], preferred_element_type=_F32)
    ch = q.shape[0]
    h1 = jax.nn.relu(a[None, :, :] - bq[:, None, :])
    h1 = h1.reshape(ch * P, 64)
    h2 = jax.nn.relu(jnp.dot(h1, w2_ref[...], preferred_element_type=_F32)
                     + b2_ref[...])
    h3 = (jnp.dot(h2, w3_ref[...], preferred_element_type=_F32)
          + b3_ref[...]).reshape(ch, P, 128)
    penalty = jnp.where(mask, 0.0, -jnp.inf).astype(_F32)
    out_ref[0] = jnp.max(h3 + penalty[:, :, None], axis=1)


def _sa1(pos3, pos1, sa1_params):
    (w1, b1), (w2, b2), (w3, b3) = sa1_params
    ch = 32
    return pl.pallas_call(
        _sa1_kernel,
        grid=(B, N1 // ch),
        in_specs=[
            pl.BlockSpec((1, P, 3), lambda c, s: (c, 0, 0)),
            pl.BlockSpec((1, ch, 3), lambda c, s: (c, s, 0)),
            pl.BlockSpec((3, 64), lambda c, s: (0, 0)),
            pl.BlockSpec((1, 64), lambda c, s: (0, 0)),
            pl.BlockSpec((64, 64), lambda c, s: (0, 0)),
            pl.BlockSpec((1, 64), lambda c, s: (0, 0)),
            pl.BlockSpec((64, 128), lambda c, s: (0, 0)),
            pl.BlockSpec((1, 128), lambda c, s: (0, 0)),
        ],
        out_specs=pl.BlockSpec((1, ch, 128), lambda c, s: (c, s, 0)),
        out_shape=jax.ShapeDtypeStruct((B, N1, 128), _F32),
    )(pos3, pos1, w1, b1[None], w2, b2[None], w3, b3[None])


def _sa2_kernel(p1_ref, ctr_ref, x1_ref, w1x_ref, w1r_ref, b1_ref,
                w2_ref, b2_ref, w3_ref, b3_ref, out_ref):
    p = p1_ref[0]
    q = ctr_ref[0]
    x1 = x1_ref[0]
    np_ = jnp.sum(p * p, axis=1)
    nq = jnp.sum(q * q, axis=1)
    d2 = nq[:, None] + np_[None, :] - 2.0 * jnp.dot(
        q, p.T, preferred_element_type=_F32)
    t = _nth_smallest(d2, MAXN, R2SQ, 40)
    mask = d2 <= t
    shared = (jnp.dot(x1, w1x_ref[...], preferred_element_type=_F32)
              + jnp.dot(p, w1r_ref[...], preferred_element_type=_F32)
              + b1_ref[...])
    qr = jnp.dot(q, w1r_ref[...], preferred_element_type=_F32)
    ch = q.shape[0]
    h1 = jax.nn.relu(shared[None, :, :] - qr[:, None, :]).reshape(ch * N1, 128)
    h2 = jax.nn.relu(jnp.dot(h1, w2_ref[...], preferred_element_type=_F32)
                     + b2_ref[...])
    h3 = (jnp.dot(h2, w3_ref[...], preferred_element_type=_F32)
          + b3_ref[...]).reshape(ch, N1, 256)
    penalty = jnp.where(mask, 0.0, -jnp.inf).astype(_F32)
    out_ref[0] = jnp.max(h3 + penalty[:, :, None], axis=1)


def _sa2(pos1, pos2, x1, sa2_params):
    (w1, b1), (w2, b2), (w3, b3) = sa2_params
    w1x, w1r = w1[:128], w1[128:]
    ch = 32
    return pl.pallas_call(
        _sa2_kernel,
        grid=(B, N2 // ch),
        in_specs=[
            pl.BlockSpec((1, N1, 3), lambda c, s: (c, 0, 0)),
            pl.BlockSpec((1, ch, 3), lambda c, s: (c, s, 0)),
            pl.BlockSpec((1, N1, 128), lambda c, s: (c, 0, 0)),
            pl.BlockSpec((128, 128), lambda c, s: (0, 0)),
            pl.BlockSpec((3, 128), lambda c, s: (0, 0)),
            pl.BlockSpec((1, 128), lambda c, s: (0, 0)),
            pl.BlockSpec((128, 128), lambda c, s: (0, 0)),
            pl.BlockSpec((1, 128), lambda c, s: (0, 0)),
            pl.BlockSpec((128, 256), lambda c, s: (0, 0)),
            pl.BlockSpec((1, 256), lambda c, s: (0, 0)),
        ],
        out_specs=pl.BlockSpec((1, ch, 256), lambda c, s: (c, s, 0)),
        out_shape=jax.ShapeDtypeStruct((B, N2, 256), _F32),
    )(pos1, pos2, x1, w1x, w1r, b1[None], w2, b2[None], w3, b3[None])


def _head_kernel(x2_ref, p2_ref, w1a_ref, w1b_ref, b1_ref, w2_ref, b2_ref,
                 w3_ref, b3_ref, h1_ref, c1_ref, h2_ref, c2_ref,
                 h3_ref, c3_ref, out_ref):
    x2 = x2_ref[...].reshape(B * N2, 256)
    p2 = p2_ref[...].reshape(B * N2, 3)
    g = jax.nn.relu(jnp.dot(x2, w1a_ref[...], preferred_element_type=_F32)
                    + jnp.dot(p2, w1b_ref[...], preferred_element_type=_F32)
                    + b1_ref[...])
    g = jax.nn.relu(jnp.dot(g, w2_ref[...], preferred_element_type=_F32)
                    + b2_ref[...])
    g = (jnp.dot(g, w3_ref[...], preferred_element_type=_F32)
         + b3_ref[...]).reshape(B, N2, 1024)
    feats = jnp.max(g, axis=1)
    l1 = jax.nn.relu(jnp.dot(feats, h1_ref[...], preferred_element_type=_F32)
                     + c1_ref[...])
    l2 = jax.nn.relu(jnp.dot(l1, h2_ref[...], preferred_element_type=_F32)
                     + c2_ref[...])
    logits = (jnp.dot(l2, h3_ref[...], preferred_element_type=_F32)
              + c3_ref[...])
    mx = jnp.max(logits, axis=1, keepdims=True)
    sh = logits - mx
    out_ref[...] = sh - jnp.log(jnp.sum(jnp.exp(sh), axis=1, keepdims=True))


def _head(x2, pos2, sa3_params, head_params):
    (w1, b1), (w2, b2), (w3, b3) = sa3_params
    (h1, c1), (h2, c2), (h3, c3) = head_params
    w1a, w1b = w1[:256], w1[256:]
    return pl.pallas_call(
        _head_kernel,
        out_shape=jax.ShapeDtypeStruct((B, 10), _F32),
    )(x2, pos2, w1a, w1b, b1[None], w2, b2[None], w3, b3[None],
      h1, c1[None], h2, c2[None], h3, c3[None])


def kernel(pos, batch, y, params):
    del batch, y
    pos3 = pos.reshape(B, P, 3)
    cov = _curvature_cov(pos3)
    curv = _smallest_eig_ratio(cov)
    pos1, pos2 = _fps(pos3, curv)
    x1 = _sa1(pos3, pos1, params['sa1'])
    x2 = _sa2(pos1, pos2, x1, params['sa2'])
    return _head(x2, pos2, params['sa3'], params['head'])


# speedup vs baseline: 19.6527x; 2.8644x over previous
import functools

import jax
import jax.numpy as jnp
from jax.experimental import pallas as pl

B = 8
P = 1024
N1 = 512
N2 = 128
R1SQ = 0.2 * 0.2
R2SQ = 0.4 * 0.4
MAXN = 64
K_CURV = 10
CURV_SCALAR = 10.0

_F32 = jnp.float32


def _nth_smallest(d2, n, hi0, iters):
    rows = d2.shape[0]
    lo0 = jnp.zeros((rows, 1), _F32)
    hi_init = jnp.full((rows, 1), hi0, _F32)
    nf = jnp.float32(n)

    def body(_, c):
        lo, hi = c
        mid = 0.5 * (lo + hi)
        cnt = jnp.sum(jnp.where(d2 <= mid, 1.0, 0.0), axis=1, keepdims=True)
        ge = cnt >= nf
        return jnp.where(ge, lo, mid), jnp.where(ge, mid, hi)

    _, hi = jax.lax.fori_loop(0, iters, body, (lo0, hi_init))
    return hi


def _cov_kernel(pos_ref, cov_ref):
    p = pos_ref[0]
    g = jnp.dot(p, p.T, preferred_element_type=_F32)
    n = jnp.sum(p * p, axis=1)
    d2 = n[:, None] + n[None, :] - 2.0 * g
    t = _nth_smallest(d2, K_CURV, 3.0, 44)
    m = jnp.where(d2 <= t, 1.0, 0.0)
    s1 = jnp.dot(m, p, preferred_element_type=_F32)
    mean = s1 / K_CURV
    x, y, z = p[:, 0], p[:, 1], p[:, 2]

    def s2(col):
        return jnp.sum(m * col[None, :], axis=1) / K_CURV

    c00 = s2(x * x) - mean[:, 0] * mean[:, 0]
    c11 = s2(y * y) - mean[:, 1] * mean[:, 1]
    c22 = s2(z * z) - mean[:, 2] * mean[:, 2]
    c01 = s2(x * y) - mean[:, 0] * mean[:, 1]
    c02 = s2(x * z) - mean[:, 0] * mean[:, 2]
    c12 = s2(y * z) - mean[:, 1] * mean[:, 2]
    cov_ref[0, 0, :] = c00
    cov_ref[0, 1, :] = c11
    cov_ref[0, 2, :] = c22
    cov_ref[0, 3, :] = c01
    cov_ref[0, 4, :] = c02
    cov_ref[0, 5, :] = c12
    cov_ref[0, 6, :] = jnp.zeros((P,), _F32)
    cov_ref[0, 7, :] = jnp.zeros((P,), _F32)


def _curvature_cov(pos3):
    return pl.pallas_call(
        _cov_kernel,
        grid=(B,),
        in_specs=[pl.BlockSpec((1, P, 3), lambda c: (c, 0, 0))],
        out_specs=pl.BlockSpec((1, 8, P), lambda c: (c, 0, 0)),
        out_shape=jax.ShapeDtypeStruct((B, 8, P), _F32),
    )(pos3)


def _smallest_eig_ratio(cov):
    a, b, c = cov[:, 0], cov[:, 1], cov[:, 2]
    d, e, f = cov[:, 3], cov[:, 4], cov[:, 5]
    tr = a + b + c
    q = tr / 3.0
    p1 = d * d + e * e + f * f
    a1, b1, c1 = a - q, b - q, c - q
    p2 = a1 * a1 + b1 * b1 + c1 * c1 + 2.0 * p1
    degen = p2 < 1e-30
    p = jnp.sqrt(jnp.where(degen, 1.0, p2) / 6.0)
    det = (a1 * (b1 * c1 - f * f)
           - d * (d * c1 - f * e)
           + e * (d * f - b1 * e))
    r = jnp.clip(det / (2.0 * p * p * p), -1.0, 1.0)
    phi = jnp.arccos(r) / 3.0
    eig_min = q + 2.0 * p * jnp.cos(phi + 2.0 * jnp.pi / 3.0)
    eig_min = jnp.where(degen, q, eig_min)
    return eig_min / (tr + 1e-12)


def _fps_kernel(px_ref, py_ref, pz_ref, w_ref, sel1_ref, sel2_ref):
    px, py, pz, w = px_ref[...], py_ref[...], pz_ref[...], w_ref[...]
    inf = jnp.float32(jnp.inf)

    def extract(m, plane):
        return jnp.min(jnp.where(m, plane, inf), axis=1, keepdims=True)

    def body1(i, c):
        mind, lx, ly, lz, lw = c
        vals = jnp.concatenate([lx, ly, lz, lw], axis=1)
        sel1_ref[pl.ds(i - 1, 1)] = jnp.transpose(vals)[None]
        dx, dy, dz = px - lx, py - ly, pz - lz
        mind = jnp.minimum(mind, dx * dx + dy * dy + dz * dz)
        score = mind * w
        m = score == jnp.max(score, axis=1, keepdims=True)
        return (mind, extract(m, px), extract(m, py), extract(m, pz),
                extract(m, w))

    c0 = (jnp.full((B, P), inf, _F32),
          px[:, 0:1], py[:, 0:1], pz[:, 0:1], w[:, 0:1])
    _, lx, ly, lz, lw = jax.lax.fori_loop(1, N1, body1, c0)
    vals = jnp.concatenate([lx, ly, lz, lw], axis=1)
    sel1_ref[pl.ds(N1 - 1, 1)] = jnp.transpose(vals)[None]

    s1 = sel1_ref[...]
    qx = jnp.transpose(s1[:, 0, :])
    qy = jnp.transpose(s1[:, 1, :])
    qz = jnp.transpose(s1[:, 2, :])
    qw = jnp.transpose(s1[:, 3, :])

    def body2(i, c):
        mind, lx, ly, lz = c
        vals = jnp.concatenate([lx, ly, lz], axis=1)
        sel2_ref[pl.ds(i - 1, 1)] = jnp.transpose(vals)[None]
        dx, dy, dz = qx - lx, qy - ly, qz - lz
        mind = jnp.minimum(mind, dx * dx + dy * dy + dz * dz)
        score = mind * qw
        m = score == jnp.max(score, axis=1, keepdims=True)
        return (mind, extract(m, qx), extract(m, qy), extract(m, qz))

    c0b = (jnp.full((B, N1), inf, _F32), qx[:, 0:1], qy[:, 0:1], qz[:, 0:1])
    _, lx, ly, lz = jax.lax.fori_loop(1, N2, body2, c0b)
    vals = jnp.concatenate([lx, ly, lz], axis=1)
    sel2_ref[pl.ds(N2 - 1, 1)] = jnp.transpose(vals)[None]


def _fps(pos3, curv):
    wfac = 1.0 + CURV_SCALAR * curv
    sel1, sel2 = pl.pallas_call(
        _fps_kernel,
        out_shape=(
            jax.ShapeDtypeStruct((N1, 4, B), _F32),
            jax.ShapeDtypeStruct((N2, 3, B), _F32),
        ),
    )(pos3[:, :, 0], pos3[:, :, 1], pos3[:, :, 2], wfac)
    pos1 = jnp.transpose(sel1[:, :3, :], (2, 0, 1))
    pos2 = jnp.transpose(sel2, (2, 0, 1))
    return pos1, pos2


def _sa1_kernel(pos_ref, ctr_ref, w1_ref, b1_ref, w2_ref, b2_ref,
                w3_ref, b3_ref, out_ref):
    p = pos_ref[0]
    q = ctr_ref[0]
    np_ = jnp.sum(p * p, axis=1)
    nq = jnp.sum(q * q, axis=1)
    d2 = nq[:, None] + np_[None, :] - 2.0 * jnp.dot(
        q, p.T, preferred_element_type=_F32)
    t = _nth_smallest(d2, MAXN, R1SQ, 40)
    mask = d2 <= t
    a = jnp.dot(p, w1_ref[...], preferred_element_type=_F32) + b1_ref[...]
    bq = jnp.dot(q, w1_ref[...], preferred_element_type=_F32)
    ch = q.shape[0]
    h1 = jax.nn.relu(a[None, :, :] - bq[:, None, :])
    h1 = h1.reshape(ch * P, 64)
    h2 = jax.nn.relu(jnp.dot(h1, w2_ref[...], preferred_element_type=_F32)
                     + b2_ref[...])
    h3 = (jnp.dot(h2, w3_ref[...], preferred_element_type=_F32)
          + b3_ref[...]).reshape(ch, P, 128)
    penalty = jnp.where(mask, 0.0, -jnp.inf).astype(_F32)
    out_ref[0] = jnp.max(h3 + penalty[:, :, None], axis=1)


def _sa1(pos3, pos1, sa1_params):
    (w1, b1), (w2, b2), (w3, b3) = sa1_params
    ch = 32
    return pl.pallas_call(
        _sa1_kernel,
        grid=(B, N1 // ch),
        in_specs=[
            pl.BlockSpec((1, P, 3), lambda c, s: (c, 0, 0)),
            pl.BlockSpec((1, ch, 3), lambda c, s: (c, s, 0)),
            pl.BlockSpec((3, 64), lambda c, s: (0, 0)),
            pl.BlockSpec((1, 64), lambda c, s: (0, 0)),
            pl.BlockSpec((64, 64), lambda c, s: (0, 0)),
            pl.BlockSpec((1, 64), lambda c, s: (0, 0)),
            pl.BlockSpec((64, 128), lambda c, s: (0, 0)),
            pl.BlockSpec((1, 128), lambda c, s: (0, 0)),
        ],
        out_specs=pl.BlockSpec((1, ch, 128), lambda c, s: (c, s, 0)),
        out_shape=jax.ShapeDtypeStruct((B, N1, 128), _F32),
    )(pos3, pos1, w1, b1[None], w2, b2[None], w3, b3[None])


def _sa2_kernel(p1_ref, ctr_ref, x1_ref, w1x_ref, w1r_ref, b1_ref,
                w2_ref, b2_ref, w3_ref, b3_ref, out_ref):
    p = p1_ref[0]
    q = ctr_ref[0]
    x1 = x1_ref[0]
    np_ = jnp.sum(p * p, axis=1)
    nq = jnp.sum(q * q, axis=1)
    d2 = nq[:, None] + np_[None, :] - 2.0 * jnp.dot(
        q, p.T, preferred_element_type=_F32)
    t = _nth_smallest(d2, MAXN, R2SQ, 40)
    mask = d2 <= t
    shared = (jnp.dot(x1, w1x_ref[...], preferred_element_type=_F32)
              + jnp.dot(p, w1r_ref[...], preferred_element_type=_F32)
              + b1_ref[...])
    qr = jnp.dot(q, w1r_ref[...], preferred_element_type=_F32)
    ch = q.shape[0]
    h1 = jax.nn.relu(shared[None, :, :] - qr[:, None, :]).reshape(ch * N1, 128)
    h2 = jax.nn.relu(jnp.dot(h1, w2_ref[...], preferred_element_type=_F32)
                     + b2_ref[...])
    h3 = (jnp.dot(h2, w3_ref[...], preferred_element_type=_F32)
          + b3_ref[...]).reshape(ch, N1, 256)
    penalty = jnp.where(mask, 0.0, -jnp.inf).astype(_F32)
    out_ref[0] = jnp.max(h3 + penalty[:, :, None], axis=1)


def _sa2(pos1, pos2, x1, sa2_params):
    (w1, b1), (w2, b2), (w3, b3) = sa2_params
    w1x, w1r = w1[:128], w1[128:]
    ch = 32
    return pl.pallas_call(
        _sa2_kernel,
        grid=(B, N2 // ch),
        in_specs=[
            pl.BlockSpec((1, N1, 3), lambda c, s: (c, 0, 0)),
            pl.BlockSpec((1, ch, 3), lambda c, s: (c, s, 0)),
            pl.BlockSpec((1, N1, 128), lambda c, s: (c, 0, 0)),
            pl.BlockSpec((128, 128), lambda c, s: (0, 0)),
            pl.BlockSpec((3, 128), lambda c, s: (0, 0)),
            pl.BlockSpec((1, 128), lambda c, s: (0, 0)),
            pl.BlockSpec((128, 128), lambda c, s: (0, 0)),
            pl.BlockSpec((1, 128), lambda c, s: (0, 0)),
            pl.BlockSpec((128, 256), lambda c, s: (0, 0)),
            pl.BlockSpec((1, 256), lambda c, s: (0, 0)),
        ],
        out_specs=pl.BlockSpec((1, ch, 256), lambda c, s: (c, s, 0)),
        out_shape=jax.ShapeDtypeStruct((B, N2, 256), _F32),
    )(pos1, pos2, x1, w1x, w1r, b1[None], w2, b2[None], w3, b3[None])


def _head_kernel(x2_ref, p2_ref, w1a_ref, w1b_ref, b1_ref, w2_ref, b2_ref,
                 w3_ref, b3_ref, h1_ref, c1_ref, h2_ref, c2_ref,
                 h3_ref, c3_ref, out_ref):
    x2 = x2_ref[...].reshape(B * N2, 256)
    p2 = p2_ref[...].reshape(B * N2, 3)
    g = jax.nn.relu(jnp.dot(x2, w1a_ref[...], preferred_element_type=_F32)
                    + jnp.dot(p2, w1b_ref[...], preferred_element_type=_F32)
                    + b1_ref[...])
    g = jax.nn.relu(jnp.dot(g, w2_ref[...], preferred_element_type=_F32)
                    + b2_ref[...])
    g = (jnp.dot(g, w3_ref[...], preferred_element_type=_F32)
         + b3_ref[...]).reshape(B, N2, 1024)
    feats = jnp.max(g, axis=1)
    l1 = jax.nn.relu(jnp.dot(feats, h1_ref[...], preferred_element_type=_F32)
                     + c1_ref[...])
    l2 = jax.nn.relu(jnp.dot(l1, h2_ref[...], preferred_element_type=_F32)
                     + c2_ref[...])
    logits = (jnp.dot(l2, h3_ref[...], preferred_element_type=_F32)
              + c3_ref[...])
    mx = jnp.max(logits, axis=1, keepdims=True)
    sh = logits - mx
    out_ref[...] = sh - jnp.log(jnp.sum(jnp.exp(sh), axis=1, keepdims=True))


def _head(x2, pos2, sa3_params, head_params):
    (w1, b1), (w2, b2), (w3, b3) = sa3_params
    (h1, c1), (h2, c2), (h3, c3) = head_params
    w1a, w1b = w1[:256], w1[256:]
    return pl.pallas_call(
        _head_kernel,
        out_shape=jax.ShapeDtypeStruct((B, 10), _F32),
    )(x2, pos2, w1a, w1b, b1[None], w2, b2[None], w3, b3[None],
      h1, c1[None], h2, c2[None], h3, c3[None])


def kernel(pos, batch, y, params):
    del batch, y
    pos3 = pos.reshape(B, P, 3)
    cov = _curvature_cov(pos3)
    curv = _smallest_eig_ratio(cov)
    pos1, pos2 = _fps(pos3, curv)
    x1 = _sa1(pos3, pos1, params['sa1'])
    x2 = _sa2(pos1, pos2, x1, params['sa2'])
    return _head(x2, pos2, params['sa3'], params['head'])


# speedup vs baseline: 21.6227x; 1.1002x over previous
import functools

import jax
import jax.numpy as jnp
from jax.experimental import pallas as pl

B = 8
P = 1024
N1 = 512
N2 = 128
R1SQ = 0.2 * 0.2
R2SQ = 0.4 * 0.4
MAXN = 64
K_CURV = 10
CURV_SCALAR = 10.0

_F32 = jnp.float32


def _nth_smallest(d2, n, hi0, iters):
    rows = d2.shape[0]
    lo0 = jnp.zeros((rows, 1), _F32)
    hi_init = jnp.full((rows, 1), hi0, _F32)
    nf = jnp.float32(n)

    def body(_, c):
        lo, hi = c
        mid = 0.5 * (lo + hi)
        cnt = jnp.sum(jnp.where(d2 <= mid, 1.0, 0.0), axis=1, keepdims=True)
        ge = cnt >= nf
        return jnp.where(ge, lo, mid), jnp.where(ge, mid, hi)

    _, hi = jax.lax.fori_loop(0, iters, body, (lo0, hi_init))
    return hi


def _cov_kernel(pos_ref, cov_ref):
    p = pos_ref[0]
    g = jnp.dot(p, p.T, preferred_element_type=_F32)
    n = jnp.sum(p * p, axis=1)
    d2 = n[:, None] + n[None, :] - 2.0 * g
    inf = jnp.float32(jnp.inf)

    def ext(_, t):
        return jnp.min(jnp.where(d2 > t, d2, inf), axis=1, keepdims=True)

    t = jax.lax.fori_loop(
        0, K_CURV, ext, jnp.full((P, 1), -inf, _F32))
    m = jnp.where(d2 <= t, 1.0, 0.0)
    s1 = jnp.dot(m, p, preferred_element_type=_F32)
    mean = s1 / K_CURV
    x, y, z = p[:, 0], p[:, 1], p[:, 2]

    def s2(col):
        return jnp.sum(m * col[None, :], axis=1) / K_CURV

    c00 = s2(x * x) - mean[:, 0] * mean[:, 0]
    c11 = s2(y * y) - mean[:, 1] * mean[:, 1]
    c22 = s2(z * z) - mean[:, 2] * mean[:, 2]
    c01 = s2(x * y) - mean[:, 0] * mean[:, 1]
    c02 = s2(x * z) - mean[:, 0] * mean[:, 2]
    c12 = s2(y * z) - mean[:, 1] * mean[:, 2]
    cov_ref[0, 0, :] = c00
    cov_ref[0, 1, :] = c11
    cov_ref[0, 2, :] = c22
    cov_ref[0, 3, :] = c01
    cov_ref[0, 4, :] = c02
    cov_ref[0, 5, :] = c12
    cov_ref[0, 6, :] = jnp.zeros((P,), _F32)
    cov_ref[0, 7, :] = jnp.zeros((P,), _F32)


def _curvature_cov(pos3):
    return pl.pallas_call(
        _cov_kernel,
        grid=(B,),
        in_specs=[pl.BlockSpec((1, P, 3), lambda c: (c, 0, 0))],
        out_specs=pl.BlockSpec((1, 8, P), lambda c: (c, 0, 0)),
        out_shape=jax.ShapeDtypeStruct((B, 8, P), _F32),
    )(pos3)


def _smallest_eig_ratio(cov):
    a, b, c = cov[:, 0], cov[:, 1], cov[:, 2]
    d, e, f = cov[:, 3], cov[:, 4], cov[:, 5]
    tr = a + b + c
    q = tr / 3.0
    p1 = d * d + e * e + f * f
    a1, b1, c1 = a - q, b - q, c - q
    p2 = a1 * a1 + b1 * b1 + c1 * c1 + 2.0 * p1
    degen = p2 < 1e-30
    p = jnp.sqrt(jnp.where(degen, 1.0, p2) / 6.0)
    det = (a1 * (b1 * c1 - f * f)
           - d * (d * c1 - f * e)
           + e * (d * f - b1 * e))
    r = jnp.clip(det / (2.0 * p * p * p), -1.0, 1.0)
    phi = jnp.arccos(r) / 3.0
    eig_min = q + 2.0 * p * jnp.cos(phi + 2.0 * jnp.pi / 3.0)
    eig_min = jnp.where(degen, q, eig_min)
    return eig_min / (tr + 1e-12)


def _fps_kernel(px_ref, py_ref, pz_ref, w_ref, sel1_ref, sel2_ref):
    px, py, pz, w = px_ref[...], py_ref[...], pz_ref[...], w_ref[...]
    inf = jnp.float32(jnp.inf)

    def extract(m, plane):
        return jnp.min(jnp.where(m, plane, inf), axis=1, keepdims=True)

    def body1(i, c):
        mind, lx, ly, lz, lw = c
        vals = jnp.concatenate([lx, ly, lz, lw], axis=1)
        sel1_ref[pl.ds(i - 1, 1)] = jnp.transpose(vals)[None]
        dx, dy, dz = px - lx, py - ly, pz - lz
        mind = jnp.minimum(mind, dx * dx + dy * dy + dz * dz)
        score = mind * w
        m = score == jnp.max(score, axis=1, keepdims=True)
        return (mind, extract(m, px), extract(m, py), extract(m, pz),
                extract(m, w))

    c0 = (jnp.full((B, P), inf, _F32),
          px[:, 0:1], py[:, 0:1], pz[:, 0:1], w[:, 0:1])
    _, lx, ly, lz, lw = jax.lax.fori_loop(1, N1, body1, c0)
    vals = jnp.concatenate([lx, ly, lz, lw], axis=1)
    sel1_ref[pl.ds(N1 - 1, 1)] = jnp.transpose(vals)[None]

    s1 = sel1_ref[...]
    qx = jnp.transpose(s1[:, 0, :])
    qy = jnp.transpose(s1[:, 1, :])
    qz = jnp.transpose(s1[:, 2, :])
    qw = jnp.transpose(s1[:, 3, :])

    def body2(i, c):
        mind, lx, ly, lz = c
        vals = jnp.concatenate([lx, ly, lz], axis=1)
        sel2_ref[pl.ds(i - 1, 1)] = jnp.transpose(vals)[None]
        dx, dy, dz = qx - lx, qy - ly, qz - lz
        mind = jnp.minimum(mind, dx * dx + dy * dy + dz * dz)
        score = mind * qw
        m = score == jnp.max(score, axis=1, keepdims=True)
        return (mind, extract(m, qx), extract(m, qy), extract(m, qz))

    c0b = (jnp.full((B, N1), inf, _F32), qx[:, 0:1], qy[:, 0:1], qz[:, 0:1])
    _, lx, ly, lz = jax.lax.fori_loop(1, N2, body2, c0b)
    vals = jnp.concatenate([lx, ly, lz], axis=1)
    sel2_ref[pl.ds(N2 - 1, 1)] = jnp.transpose(vals)[None]


def _fps(pos3, curv):
    wfac = 1.0 + CURV_SCALAR * curv
    sel1, sel2 = pl.pallas_call(
        _fps_kernel,
        out_shape=(
            jax.ShapeDtypeStruct((N1, 4, B), _F32),
            jax.ShapeDtypeStruct((N2, 3, B), _F32),
        ),
    )(pos3[:, :, 0], pos3[:, :, 1], pos3[:, :, 2], wfac)
    pos1 = jnp.transpose(sel1[:, :3, :], (2, 0, 1))
    pos2 = jnp.transpose(sel2, (2, 0, 1))
    return pos1, pos2


def _sa1_kernel(pos_ref, ctr_ref, w1_ref, b1_ref, w2_ref, b2_ref,
                w3_ref, b3_ref, out_ref):
    p = pos_ref[0]
    q = ctr_ref[0]
    np_ = jnp.sum(p * p, axis=1)
    nq = jnp.sum(q * q, axis=1)
    d2 = nq[:, None] + np_[None, :] - 2.0 * jnp.dot(
        q, p.T, preferred_element_type=_F32)
    t = _nth_smallest(d2, MAXN, R1SQ, 40)
    mask = d2 <= t
    a = jnp.dot(p, w1_ref[...], preferred_element_type=_F32) + b1_ref[...]
    bq = jnp.dot(q, w1_ref[...], preferred_element_type=_F32)
    ch = q.shape[0]
    h1 = jax.nn.relu(a[None, :, :] - bq[:, None, :])
    h1 = h1.reshape(ch * P, 64).astype(jnp.bfloat16)
    w2b = w2_ref[...].astype(jnp.bfloat16)
    w3b = w3_ref[...].astype(jnp.bfloat16)
    h2 = jax.nn.relu(jnp.dot(h1, w2b, preferred_element_type=_F32)
                     + b2_ref[...]).astype(jnp.bfloat16)
    h3 = (jnp.dot(h2, w3b, preferred_element_type=_F32)
          + b3_ref[...]).reshape(ch, P, 128)
    penalty = jnp.where(mask, 0.0, -jnp.inf).astype(_F32)
    out_ref[0] = jnp.max(h3 + penalty[:, :, None], axis=1)


def _sa1(pos3, pos1, sa1_params):
    (w1, b1), (w2, b2), (w3, b3) = sa1_params
    ch = 32
    return pl.pallas_call(
        _sa1_kernel,
        grid=(B, N1 // ch),
        in_specs=[
            pl.BlockSpec((1, P, 3), lambda c, s: (c, 0, 0)),
            pl.BlockSpec((1, ch, 3), lambda c, s: (c, s, 0)),
            pl.BlockSpec((3, 64), lambda c, s: (0, 0)),
            pl.BlockSpec((1, 64), lambda c, s: (0, 0)),
            pl.BlockSpec((64, 64), lambda c, s: (0, 0)),
            pl.BlockSpec((1, 64), lambda c, s: (0, 0)),
            pl.BlockSpec((64, 128), lambda c, s: (0, 0)),
            pl.BlockSpec((1, 128), lambda c, s: (0, 0)),
        ],
        out_specs=pl.BlockSpec((1, ch, 128), lambda c, s: (c, s, 0)),
        out_shape=jax.ShapeDtypeStruct((B, N1, 128), _F32),
    )(pos3, pos1, w1, b1[None], w2, b2[None], w3, b3[None])


def _sa2_kernel(p1_ref, ctr_ref, x1_ref, w1x_ref, w1r_ref, b1_ref,
                w2_ref, b2_ref, w3_ref, b3_ref, out_ref):
    p = p1_ref[0]
    q = ctr_ref[0]
    x1 = x1_ref[0]
    np_ = jnp.sum(p * p, axis=1)
    nq = jnp.sum(q * q, axis=1)
    d2 = nq[:, None] + np_[None, :] - 2.0 * jnp.dot(
        q, p.T, preferred_element_type=_F32)
    t = _nth_smallest(d2, MAXN, R2SQ, 40)
    mask = d2 <= t
    shared = (jnp.dot(x1, w1x_ref[...], preferred_element_type=_F32)
              + jnp.dot(p, w1r_ref[...], preferred_element_type=_F32)
              + b1_ref[...])
    qr = jnp.dot(q, w1r_ref[...], preferred_element_type=_F32)
    ch = q.shape[0]
    h1 = jax.nn.relu(shared[None, :, :] - qr[:, None, :]
                     ).reshape(ch * N1, 128).astype(jnp.bfloat16)
    w2b = w2_ref[...].astype(jnp.bfloat16)
    w3b = w3_ref[...].astype(jnp.bfloat16)
    h2 = jax.nn.relu(jnp.dot(h1, w2b, preferred_element_type=_F32)
                     + b2_ref[...]).astype(jnp.bfloat16)
    h3 = (jnp.dot(h2, w3b, preferred_element_type=_F32)
          + b3_ref[...]).reshape(ch, N1, 256)
    penalty = jnp.where(mask, 0.0, -jnp.inf).astype(_F32)
    out_ref[0] = jnp.max(h3 + penalty[:, :, None], axis=1)


def _sa2(pos1, pos2, x1, sa2_params):
    (w1, b1), (w2, b2), (w3, b3) = sa2_params
    w1x, w1r = w1[:128], w1[128:]
    ch = 32
    return pl.pallas_call(
        _sa2_kernel,
        grid=(B, N2 // ch),
        in_specs=[
            pl.BlockSpec((1, N1, 3), lambda c, s: (c, 0, 0)),
            pl.BlockSpec((1, ch, 3), lambda c, s: (c, s, 0)),
            pl.BlockSpec((1, N1, 128), lambda c, s: (c, 0, 0)),
            pl.BlockSpec((128, 128), lambda c, s: (0, 0)),
            pl.BlockSpec((3, 128), lambda c, s: (0, 0)),
            pl.BlockSpec((1, 128), lambda c, s: (0, 0)),
            pl.BlockSpec((128, 128), lambda c, s: (0, 0)),
            pl.BlockSpec((1, 128), lambda c, s: (0, 0)),
            pl.BlockSpec((128, 256), lambda c, s: (0, 0)),
            pl.BlockSpec((1, 256), lambda c, s: (0, 0)),
        ],
        out_specs=pl.BlockSpec((1, ch, 256), lambda c, s: (c, s, 0)),
        out_shape=jax.ShapeDtypeStruct((B, N2, 256), _F32),
    )(pos1, pos2, x1, w1x, w1r, b1[None], w2, b2[None], w3, b3[None])


def _head_kernel(x2_ref, p2_ref, w1a_ref, w1b_ref, b1_ref, w2_ref, b2_ref,
                 w3_ref, b3_ref, h1_ref, c1_ref, h2_ref, c2_ref,
                 h3_ref, c3_ref, out_ref):
    x2 = x2_ref[...].reshape(B * N2, 256)
    p2 = p2_ref[...].reshape(B * N2, 3)
    g = jax.nn.relu(jnp.dot(x2, w1a_ref[...], preferred_element_type=_F32)
                    + jnp.dot(p2, w1b_ref[...], preferred_element_type=_F32)
                    + b1_ref[...])
    g = jax.nn.relu(jnp.dot(g, w2_ref[...], preferred_element_type=_F32)
                    + b2_ref[...])
    g = (jnp.dot(g, w3_ref[...], preferred_element_type=_F32)
         + b3_ref[...]).reshape(B, N2, 1024)
    feats = jnp.max(g, axis=1)
    l1 = jax.nn.relu(jnp.dot(feats, h1_ref[...], preferred_element_type=_F32)
                     + c1_ref[...])
    l2 = jax.nn.relu(jnp.dot(l1, h2_ref[...], preferred_element_type=_F32)
                     + c2_ref[...])
    logits = (jnp.dot(l2, h3_ref[...], preferred_element_type=_F32)
              + c3_ref[...])
    mx = jnp.max(logits, axis=1, keepdims=True)
    sh = logits - mx
    out_ref[...] = sh - jnp.log(jnp.sum(jnp.exp(sh), axis=1, keepdims=True))


def _head(x2, pos2, sa3_params, head_params):
    (w1, b1), (w2, b2), (w3, b3) = sa3_params
    (h1, c1), (h2, c2), (h3, c3) = head_params
    w1a, w1b = w1[:256], w1[256:]
    return pl.pallas_call(
        _head_kernel,
        out_shape=jax.ShapeDtypeStruct((B, 10), _F32),
    )(x2, pos2, w1a, w1b, b1[None], w2, b2[None], w3, b3[None],
      h1, c1[None], h2, c2[None], h3, c3[None])


def kernel(pos, batch, y, params):
    del batch, y
    pos3 = pos.reshape(B, P, 3)
    cov = _curvature_cov(pos3)
    curv = _smallest_eig_ratio(cov)
    pos1, pos2 = _fps(pos3, curv)
    x1 = _sa1(pos3, pos1, params['sa1'])
    x2 = _sa2(pos1, pos2, x1, params['sa2'])
    return _head(x2, pos2, params['sa3'], params['head'])


# speedup vs baseline: 21.9514x; 1.0152x over previous
import functools

import jax
import jax.numpy as jnp
from jax.experimental import pallas as pl

B = 8
P = 1024
N1 = 512
N2 = 128
R1SQ = 0.2 * 0.2
R2SQ = 0.4 * 0.4
MAXN = 64
K_CURV = 10
CURV_SCALAR = 10.0

_F32 = jnp.float32


def _nth_smallest(d2, n, hi0, iters):
    rows = d2.shape[0]
    lo0 = jnp.zeros((rows, 1), _F32)
    hi_init = jnp.full((rows, 1), hi0, _F32)
    nf = jnp.float32(n)

    def body(_, c):
        lo, hi = c
        mid = 0.5 * (lo + hi)
        cnt = jnp.sum(jnp.where(d2 <= mid, 1.0, 0.0), axis=1, keepdims=True)
        ge = cnt >= nf
        return jnp.where(ge, lo, mid), jnp.where(ge, mid, hi)

    _, hi = jax.lax.fori_loop(0, iters, body, (lo0, hi_init))
    return hi


def _cov_kernel(pos_ref, cov_ref):
    p = pos_ref[0]
    g = jnp.dot(p, p.T, preferred_element_type=_F32)
    n = jnp.sum(p * p, axis=1)
    d2 = n[:, None] + n[None, :] - 2.0 * g
    inf = jnp.float32(jnp.inf)

    def ext(_, t):
        return jnp.min(jnp.where(d2 > t, d2, inf), axis=1, keepdims=True)

    t = jax.lax.fori_loop(
        0, K_CURV, ext, jnp.full((P, 1), -inf, _F32))
    m = jnp.where(d2 <= t, 1.0, 0.0)
    s1 = jnp.dot(m, p, preferred_element_type=_F32)
    mean = s1 / K_CURV
    x, y, z = p[:, 0], p[:, 1], p[:, 2]

    def s2(col):
        return jnp.sum(m * col[None, :], axis=1) / K_CURV

    c00 = s2(x * x) - mean[:, 0] * mean[:, 0]
    c11 = s2(y * y) - mean[:, 1] * mean[:, 1]
    c22 = s2(z * z) - mean[:, 2] * mean[:, 2]
    c01 = s2(x * y) - mean[:, 0] * mean[:, 1]
    c02 = s2(x * z) - mean[:, 0] * mean[:, 2]
    c12 = s2(y * z) - mean[:, 1] * mean[:, 2]
    cov_ref[0, 0, :] = c00
    cov_ref[0, 1, :] = c11
    cov_ref[0, 2, :] = c22
    cov_ref[0, 3, :] = c01
    cov_ref[0, 4, :] = c02
    cov_ref[0, 5, :] = c12
    cov_ref[0, 6, :] = jnp.zeros((P,), _F32)
    cov_ref[0, 7, :] = jnp.zeros((P,), _F32)


def _curvature_cov(pos3):
    return pl.pallas_call(
        _cov_kernel,
        grid=(B,),
        in_specs=[pl.BlockSpec((1, P, 3), lambda c: (c, 0, 0))],
        out_specs=pl.BlockSpec((1, 8, P), lambda c: (c, 0, 0)),
        out_shape=jax.ShapeDtypeStruct((B, 8, P), _F32),
    )(pos3)


def _smallest_eig_ratio(cov):
    a, b, c = cov[:, 0], cov[:, 1], cov[:, 2]
    d, e, f = cov[:, 3], cov[:, 4], cov[:, 5]
    tr = a + b + c
    q = tr / 3.0
    p1 = d * d + e * e + f * f
    a1, b1, c1 = a - q, b - q, c - q
    p2 = a1 * a1 + b1 * b1 + c1 * c1 + 2.0 * p1
    degen = p2 < 1e-30
    p = jnp.sqrt(jnp.where(degen, 1.0, p2) / 6.0)
    det = (a1 * (b1 * c1 - f * f)
           - d * (d * c1 - f * e)
           + e * (d * f - b1 * e))
    r = jnp.clip(det / (2.0 * p * p * p), -1.0, 1.0)
    phi = jnp.arccos(r) / 3.0
    eig_min = q + 2.0 * p * jnp.cos(phi + 2.0 * jnp.pi / 3.0)
    eig_min = jnp.where(degen, q, eig_min)
    return eig_min / (tr + 1e-12)


def _fps_kernel(px_ref, py_ref, pz_ref, w_ref, sel1_ref, sel2_ref):
    px, py, pz, w = px_ref[...], py_ref[...], pz_ref[...], w_ref[...]
    inf = jnp.float32(jnp.inf)

    def extract(m, plane):
        return jnp.min(jnp.where(m, plane, inf), axis=1, keepdims=True)

    def body1(i, c):
        mind, lx, ly, lz, lw = c
        vals = jnp.concatenate([lx, ly, lz, lw], axis=1)
        sel1_ref[pl.ds(i - 1, 1)] = jnp.transpose(vals)[None]
        dx, dy, dz = px - lx, py - ly, pz - lz
        mind = jnp.minimum(mind, dx * dx + dy * dy + dz * dz)
        score = mind * w
        m = score == jnp.max(score, axis=1, keepdims=True)
        return (mind, extract(m, px), extract(m, py), extract(m, pz),
                extract(m, w))

    c0 = (jnp.full((B, P), inf, _F32),
          px[:, 0:1], py[:, 0:1], pz[:, 0:1], w[:, 0:1])
    _, lx, ly, lz, lw = jax.lax.fori_loop(1, N1, body1, c0)
    vals = jnp.concatenate([lx, ly, lz, lw], axis=1)
    sel1_ref[pl.ds(N1 - 1, 1)] = jnp.transpose(vals)[None]

    s1 = sel1_ref[...]
    qx = jnp.transpose(s1[:, 0, :])
    qy = jnp.transpose(s1[:, 1, :])
    qz = jnp.transpose(s1[:, 2, :])
    qw = jnp.transpose(s1[:, 3, :])

    def body2(i, c):
        mind, lx, ly, lz = c
        vals = jnp.concatenate([lx, ly, lz], axis=1)
        sel2_ref[pl.ds(i - 1, 1)] = jnp.transpose(vals)[None]
        dx, dy, dz = qx - lx, qy - ly, qz - lz
        mind = jnp.minimum(mind, dx * dx + dy * dy + dz * dz)
        score = mind * qw
        m = score == jnp.max(score, axis=1, keepdims=True)
        return (mind, extract(m, qx), extract(m, qy), extract(m, qz))

    c0b = (jnp.full((B, N1), inf, _F32), qx[:, 0:1], qy[:, 0:1], qz[:, 0:1])
    _, lx, ly, lz = jax.lax.fori_loop(1, N2, body2, c0b)
    vals = jnp.concatenate([lx, ly, lz], axis=1)
    sel2_ref[pl.ds(N2 - 1, 1)] = jnp.transpose(vals)[None]


def _fps(pos3, curv):
    wfac = 1.0 + CURV_SCALAR * curv
    sel1, sel2 = pl.pallas_call(
        _fps_kernel,
        out_shape=(
            jax.ShapeDtypeStruct((N1, 4, B), _F32),
            jax.ShapeDtypeStruct((N2, 3, B), _F32),
        ),
    )(pos3[:, :, 0], pos3[:, :, 1], pos3[:, :, 2], wfac)
    pos1 = jnp.transpose(sel1[:, :3, :], (2, 0, 1))
    pos2 = jnp.transpose(sel2, (2, 0, 1))
    return pos1, pos2


def _sa1_kernel(pos_ref, ctr_ref, w1_ref, b1_ref, w2_ref, b2_ref,
                w3_ref, b3_ref, out_ref):
    p = pos_ref[0]
    q = ctr_ref[0]
    np_ = jnp.sum(p * p, axis=1)
    nq = jnp.sum(q * q, axis=1)
    d2 = nq[:, None] + np_[None, :] - 2.0 * jnp.dot(
        q, p.T, preferred_element_type=_F32)
    t = _nth_smallest(d2, MAXN, R1SQ, 40)
    mask = d2 <= t
    a = (jnp.dot(p, w1_ref[...], preferred_element_type=_F32)
         + b1_ref[...]).astype(jnp.bfloat16)
    bq = jnp.dot(q, w1_ref[...],
                 preferred_element_type=_F32).astype(jnp.bfloat16)
    ch = q.shape[0]
    h1 = jax.nn.relu(a[None, :, :] - bq[:, None, :])
    h1 = h1.reshape(ch * P, 64)
    w2b = w2_ref[...].astype(jnp.bfloat16)
    w3b = w3_ref[...].astype(jnp.bfloat16)
    b2b = b2_ref[...].astype(jnp.bfloat16)
    h2 = jax.nn.relu(
        jnp.dot(h1, w2b, preferred_element_type=_F32).astype(jnp.bfloat16)
        + b2b)
    h3 = jnp.dot(h2, w3b, preferred_element_type=_F32
                 ).astype(jnp.bfloat16).reshape(ch, P, 128)
    penalty = jnp.where(mask, 0.0, -jnp.inf).astype(jnp.bfloat16)
    pooled = jnp.max(h3 + penalty[:, :, None], axis=1).astype(_F32)
    out_ref[0] = pooled + b3_ref[...]


def _sa1(pos3, pos1, sa1_params):
    (w1, b1), (w2, b2), (w3, b3) = sa1_params
    ch = 32
    return pl.pallas_call(
        _sa1_kernel,
        grid=(B, N1 // ch),
        in_specs=[
            pl.BlockSpec((1, P, 3), lambda c, s: (c, 0, 0)),
            pl.BlockSpec((1, ch, 3), lambda c, s: (c, s, 0)),
            pl.BlockSpec((3, 64), lambda c, s: (0, 0)),
            pl.BlockSpec((1, 64), lambda c, s: (0, 0)),
            pl.BlockSpec((64, 64), lambda c, s: (0, 0)),
            pl.BlockSpec((1, 64), lambda c, s: (0, 0)),
            pl.BlockSpec((64, 128), lambda c, s: (0, 0)),
            pl.BlockSpec((1, 128), lambda c, s: (0, 0)),
        ],
        out_specs=pl.BlockSpec((1, ch, 128), lambda c, s: (c, s, 0)),
        out_shape=jax.ShapeDtypeStruct((B, N1, 128), _F32),
    )(pos3, pos1, w1, b1[None], w2, b2[None], w3, b3[None])


def _sa2_kernel(p1_ref, ctr_ref, x1_ref, w1x_ref, w1r_ref, b1_ref,
                w2_ref, b2_ref, w3_ref, b3_ref, out_ref):
    p = p1_ref[0]
    q = ctr_ref[0]
    x1 = x1_ref[0]
    np_ = jnp.sum(p * p, axis=1)
    nq = jnp.sum(q * q, axis=1)
    d2 = nq[:, None] + np_[None, :] - 2.0 * jnp.dot(
        q, p.T, preferred_element_type=_F32)
    t = _nth_smallest(d2, MAXN, R2SQ, 40)
    mask = d2 <= t
    shared = (jnp.dot(x1, w1x_ref[...], preferred_element_type=_F32)
              + jnp.dot(p, w1r_ref[...], preferred_element_type=_F32)
              + b1_ref[...])
    qr = jnp.dot(q, w1r_ref[...],
                 preferred_element_type=_F32).astype(jnp.bfloat16)
    sharedb = shared.astype(jnp.bfloat16)
    ch = q.shape[0]
    h1 = jax.nn.relu(sharedb[None, :, :] - qr[:, None, :]
                     ).reshape(ch * N1, 128)
    w2b = w2_ref[...].astype(jnp.bfloat16)
    w3b = w3_ref[...].astype(jnp.bfloat16)
    b2b = b2_ref[...].astype(jnp.bfloat16)
    h2 = jax.nn.relu(
        jnp.dot(h1, w2b, preferred_element_type=_F32).astype(jnp.bfloat16)
        + b2b)
    h3 = jnp.dot(h2, w3b, preferred_element_type=_F32
                 ).astype(jnp.bfloat16).reshape(ch, N1, 256)
    penalty = jnp.where(mask, 0.0, -jnp.inf).astype(jnp.bfloat16)
    pooled = jnp.max(h3 + penalty[:, :, None], axis=1).astype(_F32)
    out_ref[0] = pooled + b3_ref[...]


def _sa2(pos1, pos2, x1, sa2_params):
    (w1, b1), (w2, b2), (w3, b3) = sa2_params
    w1x, w1r = w1[:128], w1[128:]
    ch = 32
    return pl.pallas_call(
        _sa2_kernel,
        grid=(B, N2 // ch),
        in_specs=[
            pl.BlockSpec((1, N1, 3), lambda c, s: (c, 0, 0)),
            pl.BlockSpec((1, ch, 3), lambda c, s: (c, s, 0)),
            pl.BlockSpec((1, N1, 128), lambda c, s: (c, 0, 0)),
            pl.BlockSpec((128, 128), lambda c, s: (0, 0)),
            pl.BlockSpec((3, 128), lambda c, s: (0, 0)),
            pl.BlockSpec((1, 128), lambda c, s: (0, 0)),
            pl.BlockSpec((128, 128), lambda c, s: (0, 0)),
            pl.BlockSpec((1, 128), lambda c, s: (0, 0)),
            pl.BlockSpec((128, 256), lambda c, s: (0, 0)),
            pl.BlockSpec((1, 256), lambda c, s: (0, 0)),
        ],
        out_specs=pl.BlockSpec((1, ch, 256), lambda c, s: (c, s, 0)),
        out_shape=jax.ShapeDtypeStruct((B, N2, 256), _F32),
    )(pos1, pos2, x1, w1x, w1r, b1[None], w2, b2[None], w3, b3[None])


def _head_kernel(x2_ref, p2_ref, w1a_ref, w1b_ref, b1_ref, w2_ref, b2_ref,
                 w3_ref, b3_ref, h1_ref, c1_ref, h2_ref, c2_ref,
                 h3_ref, c3_ref, out_ref):
    x2 = x2_ref[...].reshape(B * N2, 256)
    p2 = p2_ref[...].reshape(B * N2, 3)
    g = jax.nn.relu(jnp.dot(x2, w1a_ref[...], preferred_element_type=_F32)
                    + jnp.dot(p2, w1b_ref[...], preferred_element_type=_F32)
                    + b1_ref[...])
    g = jax.nn.relu(jnp.dot(g, w2_ref[...], preferred_element_type=_F32)
                    + b2_ref[...])
    g = (jnp.dot(g, w3_ref[...], preferred_element_type=_F32)
         + b3_ref[...]).reshape(B, N2, 1024)
    feats = jnp.max(g, axis=1)
    l1 = jax.nn.relu(jnp.dot(feats, h1_ref[...], preferred_element_type=_F32)
                     + c1_ref[...])
    l2 = jax.nn.relu(jnp.dot(l1, h2_ref[...], preferred_element_type=_F32)
                     + c2_ref[...])
    logits = (jnp.dot(l2, h3_ref[...], preferred_element_type=_F32)
              + c3_ref[...])
    mx = jnp.max(logits, axis=1, keepdims=True)
    sh = logits - mx
    out_ref[...] = sh - jnp.log(jnp.sum(jnp.exp(sh), axis=1, keepdims=True))


def _head(x2, pos2, sa3_params, head_params):
    (w1, b1), (w2, b2), (w3, b3) = sa3_params
    (h1, c1), (h2, c2), (h3, c3) = head_params
    w1a, w1b = w1[:256], w1[256:]
    return pl.pallas_call(
        _head_kernel,
        out_shape=jax.ShapeDtypeStruct((B, 10), _F32),
    )(x2, pos2, w1a, w1b, b1[None], w2, b2[None], w3, b3[None],
      h1, c1[None], h2, c2[None], h3, c3[None])


def kernel(pos, batch, y, params):
    del batch, y
    pos3 = pos.reshape(B, P, 3)
    cov = _curvature_cov(pos3)
    curv = _smallest_eig_ratio(cov)
    pos1, pos2 = _fps(pos3, curv)
    x1 = _sa1(pos3, pos1, params['sa1'])
    x2 = _sa2(pos1, pos2, x1, params['sa2'])
    return _head(x2, pos2, params['sa3'], params['head'])
